# Initial kernel scaffold; baseline (speedup 1.0000x reference)
import jax
import jax.numpy as jnp
from jax.experimental import pallas as pl


def kernel(pos, batch, template, W1, b1, W2, b2):
    raise NotImplementedError("write your pallas kernel here")


# speedup vs baseline: 3.0011x; 3.0011x over previous
import functools

import jax
import jax.numpy as jnp
from jax import lax
from jax.experimental import pallas as pl
from jax.experimental.pallas import tpu as pltpu
from jax.experimental.pallas import tpu_sc as plsc

N = 32768
NB = 8
T = 2048
D = 128
K = 64

BLK = 512
NBLKS = N // BLK


def _knn_body(pos_ref, tmpl_ref, w1_ref, b1_ref, w2_ref, b2_ref,
              x_ref, xt_ref, nn_ref, xta_s):
    @pl.when(pl.program_id(0) == 0)
    def _():
        h = jnp.maximum(
            jnp.dot(tmpl_ref[...], w1_ref[...],
                    preferred_element_type=jnp.float32) + b1_ref[...], 0.0)
        xt = jnp.dot(h, w2_ref[...],
                     preferred_element_type=jnp.float32) + b2_ref[...]
        xt_ref[...] = xt
        xtn = jnp.sum(xt * xt, axis=1, keepdims=True)
        xta_s[...] = jnp.concatenate([xt, xtn], axis=1)

    h = jnp.maximum(
        jnp.dot(pos_ref[...], w1_ref[...],
                preferred_element_type=jnp.float32) + b1_ref[...], 0.0)
    x = jnp.dot(h, w2_ref[...],
                preferred_element_type=jnp.float32) + b2_ref[...]
    x_ref[...] = x
    xa = jnp.concatenate([-2.0 * x, jnp.ones((BLK, 1), jnp.float32)], axis=1)
    d2 = lax.dot_general(xa, xta_s[...], (((1,), (1,)), ((), ())),
                         preferred_element_type=jnp.float32)
    m = jnp.min(d2, axis=1, keepdims=True)
    ii = lax.broadcasted_iota(jnp.int32, (BLK, T), 1)
    idx = jnp.min(jnp.where(d2 <= m, ii, T), axis=1, keepdims=True)
    nn_ref[...] = idx[None]


def _knn(pos, template, W1, b1r, W2, b2r):
    return pl.pallas_call(
        _knn_body,
        grid=(NBLKS,),
        in_specs=[
            pl.BlockSpec((BLK, 3), lambda i: (i, 0)),
            pl.BlockSpec((T, 3), lambda i: (0, 0)),
            pl.BlockSpec((3, D), lambda i: (0, 0)),
            pl.BlockSpec((1, D), lambda i: (0, 0)),
            pl.BlockSpec((D, D), lambda i: (0, 0)),
            pl.BlockSpec((1, D), lambda i: (0, 0)),
        ],
        out_specs=[
            pl.BlockSpec((BLK, D), lambda i: (i, 0)),
            pl.BlockSpec((T, D), lambda i: (0, 0)),
            pl.BlockSpec((1, BLK, 1), lambda i: (i, 0, 0)),
        ],
        out_shape=[
            jax.ShapeDtypeStruct((N, D), jnp.float32),
            jax.ShapeDtypeStruct((T, D), jnp.float32),
            jax.ShapeDtypeStruct((NBLKS, BLK, 1), jnp.int32),
        ],
        scratch_shapes=[pltpu.VMEM((T, D + 1), jnp.float32)],
    )(pos, template, W1, b1r, W2, b2r)


_NC = 2
_NS = 16
_NW = _NC * _NS
_CHUNK = N // _NW
_SUB = 128
_NSUB = _CHUNK // _SUB

@functools.lru_cache(maxsize=1)
def _sc_mesh():
    return plsc.VectorSubcoreMesh(core_axis_name="c", subcore_axis_name="s")


def _s_body(x_hbm, xt_hbm, nn_hbm, s_hbm, idx_v, rows_v, x_v, s_v, sem):
    wid = lax.axis_index("s") * _NC + lax.axis_index("c")
    base = wid * _CHUNK

    def chunk(ci, carry):
        off = base + ci * _SUB
        pltpu.sync_copy(nn_hbm.at[pl.ds(off, _SUB)], idx_v)
        cp = pltpu.async_copy(xt_hbm.at[idx_v], rows_v, sem)
        pltpu.sync_copy(x_hbm.at[pl.ds(off, _SUB)], x_v)
        cp.wait()

        lane = lax.iota(jnp.int32, 16)

        def group(g, c2):
            sacc = jnp.zeros((16,), jnp.float32)
            for r16 in range(16):
                r = g * 16 + r16
                acc = jnp.zeros((16,), jnp.float32)
                for j in range(D // 16):
                    tv = rows_v[r, pl.ds(j * 16, 16)]
                    xv = x_v[r, pl.ds(j * 16, 16)]
                    dd = tv - xv
                    acc = acc + dd * dd
                sacc = jnp.where(lane == r16, jnp.sum(acc), sacc)
            s_v[pl.ds(g * 16, 16)] = 1.0 / (1.0 + sacc)
            return c2

        lax.fori_loop(0, _SUB // 16, group, 0)
        pltpu.sync_copy(s_v, s_hbm.at[pl.ds(off, _SUB)])
        return carry

    lax.fori_loop(0, _NSUB, chunk, 0)


@functools.lru_cache(maxsize=1)
def _s_call():
    return functools.partial(
        pl.kernel,
        out_type=jax.ShapeDtypeStruct((N,), jnp.float32),
        scratch_types=[
            pltpu.VMEM((_SUB,), jnp.int32),
            pltpu.VMEM((_SUB, D), jnp.float32),
            pltpu.VMEM((_SUB, D), jnp.float32),
            pltpu.VMEM((_SUB,), jnp.float32),
            pltpu.SemaphoreType.DMA,
        ],
        mesh=_sc_mesh(),
        compiler_params=pltpu.CompilerParams(needs_layout_passes=False),
    )(_s_body)


def _topk_body(s_ref, b_ref, out_ref, keys_s):
    nidx = lax.broadcasted_iota(jnp.int32, (NB, N), 1)
    rows = lax.broadcasted_iota(jnp.int32, (NB, N), 0)
    sv = s_ref[...]
    bv = b_ref[...]
    keys_s[...] = jnp.where(bv == rows, jnp.broadcast_to(sv, (NB, N)),
                            -(nidx.astype(jnp.float32) + 1.0))

    def round_(k, acc):
        keys = keys_s[...]
        m = jnp.max(keys, axis=1, keepdims=True)
        sel = jnp.min(jnp.where(keys >= m, nidx, N), axis=1, keepdims=True)
        keys_s[...] = jnp.where(nidx == sel, -3e9, keys)
        kcol = lax.broadcasted_iota(jnp.int32, (NB, K), 1)
        return jnp.where(kcol == k, sel, acc)

    out_ref[...] = lax.fori_loop(0, K, round_, jnp.zeros((NB, K), jnp.int32))


def _topk(s2, b2):
    return pl.pallas_call(
        _topk_body,
        in_specs=[
            pl.BlockSpec((1, N), lambda: (0, 0)),
            pl.BlockSpec((1, N), lambda: (0, 0)),
        ],
        out_specs=pl.BlockSpec((NB, K), lambda: (0, 0)),
        out_shape=jax.ShapeDtypeStruct((NB, K), jnp.int32),
        scratch_shapes=[pltpu.VMEM((NB, N), jnp.float32)],
    )(s2, b2)


def _gh_body(topk_hbm, posf_hbm, tmplf_hbm, nn_hbm, g_hbm, h_hbm,
             idx_v, i3_v, nnsel_v, px_v, py_v, pz_v, tx_v, ty_v, tz_v,
             out_v, sem):
    wid = lax.axis_index("s") * _NC + lax.axis_index("c")

    @pl.when(wid < NB)
    def _():
        b = wid
        pltpu.sync_copy(topk_hbm.at[pl.ds(b * K, K)], idx_v)
        pltpu.async_copy(nn_hbm.at[idx_v], nnsel_v, sem).wait()
        for c, dest in ((0, px_v), (1, py_v), (2, pz_v)):
            for v in range(K // 16):
                sl = pl.ds(v * 16, 16)
                i3_v[sl] = idx_v[sl] * 3 + c
            pltpu.async_copy(posf_hbm.at[i3_v], dest, sem).wait()
        for c, dest in ((0, tx_v), (1, ty_v), (2, tz_v)):
            for v in range(K // 16):
                sl = pl.ds(v * 16, 16)
                i3_v[sl] = nnsel_v[sl] * 3 + c
            pltpu.async_copy(tmplf_hbm.at[i3_v], dest, sem).wait()

        ones = jnp.ones((16,), jnp.float32)

        def dotsum(ar, br):
            t = jnp.zeros((16,), jnp.float32)
            for v in range(K // 16):
                sl = pl.ds(v * 16, 16)
                av = ar[sl] if ar is not None else ones
                bv = br[sl] if br is not None else ones
                t = t + av * bv
            return jnp.sum(t)

        lane = lax.iota(jnp.int32, 16)
        cm = (px_v, py_v, pz_v, None)
        cf = (tx_v, ty_v, tz_v, None)
        gvec = jnp.zeros((16,), jnp.float32)
        hvec = jnp.zeros((16,), jnp.float32)
        for i in range(4):
            for j in range(4):
                gvec = jnp.where(lane == i * 4 + j, dotsum(cm[i], cm[j]), gvec)
                hvec = jnp.where(lane == i * 4 + j, dotsum(cm[i], cf[j]), hvec)
        out_v[...] = gvec
        pltpu.sync_copy(out_v, g_hbm.at[b])
        out_v[...] = hvec
        pltpu.sync_copy(out_v, h_hbm.at[b])


@functools.lru_cache(maxsize=1)
def _gh_call():
    return functools.partial(
        pl.kernel,
        out_type=[
            jax.ShapeDtypeStruct((NB, 16), jnp.float32),
            jax.ShapeDtypeStruct((NB, 16), jnp.float32),
        ],
        scratch_types=[
            pltpu.VMEM((K,), jnp.int32),
            pltpu.VMEM((K,), jnp.int32),
            pltpu.VMEM((K,), jnp.int32),
            pltpu.VMEM((K,), jnp.float32),
            pltpu.VMEM((K,), jnp.float32),
            pltpu.VMEM((K,), jnp.float32),
            pltpu.VMEM((K,), jnp.float32),
            pltpu.VMEM((K,), jnp.float32),
            pltpu.VMEM((K,), jnp.float32),
            pltpu.VMEM((16,), jnp.float32),
            pltpu.SemaphoreType.DMA,
        ],
        mesh=_sc_mesh(),
        compiler_params=pltpu.CompilerParams(needs_layout_passes=False),
    )(_gh_body)


BLK2 = 4096
NBLK2 = N // BLK2


def _solve_body(g_ref, h_ref, pos_ref, b_ref, out_ref):
    g = g_ref[...]
    h = h_ref[...]

    def c(mat, i, j):
        k = i * 4 + j
        return mat[:, k:k + 1]

    d0 = c(g, 0, 0)
    L10 = c(g, 1, 0) / d0
    L20 = c(g, 2, 0) / d0
    L30 = c(g, 3, 0) / d0
    d1 = c(g, 1, 1) - L10 * L10 * d0
    L21 = (c(g, 2, 1) - L20 * L10 * d0) / d1
    L31 = (c(g, 3, 1) - L30 * L10 * d0) / d1
    d2_ = c(g, 2, 2) - L20 * L20 * d0 - L21 * L21 * d1
    L32 = (c(g, 3, 2) - L30 * L20 * d0 - L31 * L21 * d1) / d2_
    d3 = (c(g, 3, 3) - L30 * L30 * d0 - L31 * L31 * d1 - L32 * L32 * d2_)

    acols = []
    for j in range(4):
        h0, h1, h2, h3 = c(h, 0, j), c(h, 1, j), c(h, 2, j), c(h, 3, j)
        y0 = h0
        y1 = h1 - L10 * y0
        y2 = h2 - L20 * y0 - L21 * y1
        y3 = h3 - L30 * y0 - L31 * y1 - L32 * y2
        z0, z1, z2, z3 = y0 / d0, y1 / d1, y2 / d2_, y3 / d3
        x3 = z3
        x2 = z2 - L32 * x3
        x1 = z1 - L21 * x2 - L31 * x3
        x0 = z0 - L10 * x1 - L20 * x2 - L30 * x3
        acols.append((x0, x1, x2, x3))
    aflat = jnp.concatenate(
        [acols[j][i] for i in range(4) for j in range(4)], axis=1)

    bb = b_ref[...]
    oh = (bb == lax.broadcasted_iota(jnp.int32, (BLK2, NB), 1)
          ).astype(jnp.float32)
    contrib = jnp.dot(oh, aflat, preferred_element_type=jnp.float32,
                      precision=lax.Precision.HIGHEST)
    p = pos_ref[...]
    outs = []
    for j in range(3):
        o = (p[:, 0:1] * contrib[:, j:j + 1]
             + p[:, 1:2] * contrib[:, 4 + j:5 + j]
             + p[:, 2:3] * contrib[:, 8 + j:9 + j]
             + contrib[:, 12 + j:13 + j])
        outs.append(o)
    out_ref[...] = jnp.concatenate(outs, axis=1)


def _solve(G, H, pos, b2):
    return pl.pallas_call(
        _solve_body,
        grid=(NBLK2,),
        in_specs=[
            pl.BlockSpec((NB, 16), lambda i: (0, 0)),
            pl.BlockSpec((NB, 16), lambda i: (0, 0)),
            pl.BlockSpec((BLK2, 3), lambda i: (i, 0)),
            pl.BlockSpec((BLK2, 1), lambda i: (i, 0)),
        ],
        out_specs=pl.BlockSpec((BLK2, 3), lambda i: (i, 0)),
        out_shape=jax.ShapeDtypeStruct((N, 3), jnp.float32),
    )(G, H, pos, b2)


def kernel(pos, batch, template, W1, b1, W2, b2):
    X, Xt, nn3 = _knn(pos, template, W1, b1.reshape(1, D), W2,
                      b2.reshape(1, D))
    nn = nn3.reshape(N)
    S = _s_call()(X, Xt, nn)
    topk = _topk(S.reshape(1, N), batch.reshape(1, N))
    G, H = _gh_call()(topk.reshape(NB * K), pos.reshape(N * 3),
                      template.reshape(T * 3), nn)
    return _solve(G, H, pos, batch.reshape(N, 1))


# speedup vs baseline: 4.6185x; 1.5389x over previous
import functools

import jax
import jax.numpy as jnp
from jax import lax
from jax.experimental import pallas as pl
from jax.experimental.pallas import tpu as pltpu
from jax.experimental.pallas import tpu_sc as plsc

N = 32768
NB = 8
T = 2048
D = 128
K = 64

BLK = 512
NBLKS = N // BLK


def _knn_body(pos_ref, tmpl_ref, w1_ref, b1_ref, w2_ref, b2_ref,
              nn_ref, s_ref, xt_s, xtn_s):
    @pl.when(pl.program_id(0) == 0)
    def _():
        h = jnp.maximum(
            jnp.dot(tmpl_ref[...], w1_ref[...],
                    preferred_element_type=jnp.float32) + b1_ref[...], 0.0)
        xt = jnp.dot(h, w2_ref[...],
                     preferred_element_type=jnp.float32) + b2_ref[...]
        xt_s[...] = xt
        xtn_s[...] = lax.dot_general(
            jnp.ones((1, D), jnp.float32), xt * xt,
            (((1,), (1,)), ((), ())), preferred_element_type=jnp.float32)

    h = jnp.maximum(
        jnp.dot(pos_ref[...], w1_ref[...],
                preferred_element_type=jnp.float32) + b1_ref[...], 0.0)
    x = jnp.dot(h, w2_ref[...],
                preferred_element_type=jnp.float32) + b2_ref[...]
    xn = jnp.sum(x * x, axis=1, keepdims=True)
    d2 = xtn_s[...] + lax.dot_general(-2.0 * x, xt_s[...],
                                      (((1,), (1,)), ((), ())),
                                      preferred_element_type=jnp.float32)
    m = jnp.min(d2, axis=1, keepdims=True)
    ii = lax.broadcasted_iota(jnp.int32, (BLK, T), 1)
    idx = jnp.min(jnp.where(d2 <= m, ii, T), axis=1, keepdims=True)
    nn_ref[...] = idx[None]
    s_ref[...] = (1.0 / (1.0 + (xn + m)))[None]


def _knn(pos, template, W1, b1r, W2, b2r):
    return pl.pallas_call(
        _knn_body,
        grid=(NBLKS,),
        in_specs=[
            pl.BlockSpec((BLK, 3), lambda i: (i, 0)),
            pl.BlockSpec((T, 3), lambda i: (0, 0)),
            pl.BlockSpec((3, D), lambda i: (0, 0)),
            pl.BlockSpec((1, D), lambda i: (0, 0)),
            pl.BlockSpec((D, D), lambda i: (0, 0)),
            pl.BlockSpec((1, D), lambda i: (0, 0)),
        ],
        out_specs=[
            pl.BlockSpec((1, BLK, 1), lambda i: (i, 0, 0)),
            pl.BlockSpec((1, BLK, 1), lambda i: (i, 0, 0)),
        ],
        out_shape=[
            jax.ShapeDtypeStruct((NBLKS, BLK, 1), jnp.int32),
            jax.ShapeDtypeStruct((NBLKS, BLK, 1), jnp.float32),
        ],
        scratch_shapes=[pltpu.VMEM((T, D), jnp.float32),
                        pltpu.VMEM((1, T), jnp.float32)],
    )(pos, template, W1, b1r, W2, b2r)


_NC = 2
_NS = 16
_NW = _NC * _NS


@functools.lru_cache(maxsize=1)
def _sc_mesh():
    return plsc.VectorSubcoreMesh(core_axis_name="c", subcore_axis_name="s")


def _topk_body(s_ref, b_ref, out_ref, keys_s):
    nidx = lax.broadcasted_iota(jnp.int32, (NB, N), 1)
    rows = lax.broadcasted_iota(jnp.int32, (NB, N), 0)
    sv = s_ref[...]
    bv = b_ref[...]
    keys_s[...] = jnp.where(bv == rows, jnp.broadcast_to(sv, (NB, N)),
                            -(nidx.astype(jnp.float32) + 1.0))

    def round_(k, acc):
        keys = keys_s[...]
        m = jnp.max(keys, axis=1, keepdims=True)
        sel = jnp.min(jnp.where(keys >= m, nidx, N), axis=1, keepdims=True)
        keys_s[...] = jnp.where(nidx == sel, -3e9, keys)
        kcol = lax.broadcasted_iota(jnp.int32, (NB, K), 1)
        return jnp.where(kcol == k, sel, acc)

    out_ref[...] = lax.fori_loop(0, K, round_, jnp.zeros((NB, K), jnp.int32))


def _topk(s2, b2):
    return pl.pallas_call(
        _topk_body,
        in_specs=[
            pl.BlockSpec((1, N), lambda: (0, 0)),
            pl.BlockSpec((1, N), lambda: (0, 0)),
        ],
        out_specs=pl.BlockSpec((NB, K), lambda: (0, 0)),
        out_shape=jax.ShapeDtypeStruct((NB, K), jnp.int32),
        scratch_shapes=[pltpu.VMEM((NB, N), jnp.float32)],
    )(s2, b2)


def _gh_body(topk_hbm, posf_hbm, tmplf_hbm, nn_hbm, g_hbm, h_hbm,
             idx_v, i3_v, nnsel_v, px_v, py_v, pz_v, tx_v, ty_v, tz_v,
             out_v, sem):
    wid = lax.axis_index("s") * _NC + lax.axis_index("c")

    @pl.when(wid < NB)
    def _():
        b = wid
        pltpu.sync_copy(topk_hbm.at[pl.ds(b * K, K)], idx_v)
        pltpu.async_copy(nn_hbm.at[idx_v], nnsel_v, sem).wait()
        for c, dest in ((0, px_v), (1, py_v), (2, pz_v)):
            for v in range(K // 16):
                sl = pl.ds(v * 16, 16)
                i3_v[sl] = idx_v[sl] * 3 + c
            pltpu.async_copy(posf_hbm.at[i3_v], dest, sem).wait()
        for c, dest in ((0, tx_v), (1, ty_v), (2, tz_v)):
            for v in range(K // 16):
                sl = pl.ds(v * 16, 16)
                i3_v[sl] = nnsel_v[sl] * 3 + c
            pltpu.async_copy(tmplf_hbm.at[i3_v], dest, sem).wait()

        ones = jnp.ones((16,), jnp.float32)

        def dotsum(ar, br):
            t = jnp.zeros((16,), jnp.float32)
            for v in range(K // 16):
                sl = pl.ds(v * 16, 16)
                av = ar[sl] if ar is not None else ones
                bv = br[sl] if br is not None else ones
                t = t + av * bv
            return jnp.sum(t)

        lane = lax.iota(jnp.int32, 16)
        cm = (px_v, py_v, pz_v, None)
        cf = (tx_v, ty_v, tz_v, None)
        gvec = jnp.zeros((16,), jnp.float32)
        hvec = jnp.zeros((16,), jnp.float32)
        for i in range(4):
            for j in range(4):
                gvec = jnp.where(lane == i * 4 + j, dotsum(cm[i], cm[j]), gvec)
                hvec = jnp.where(lane == i * 4 + j, dotsum(cm[i], cf[j]), hvec)
        out_v[...] = gvec
        pltpu.sync_copy(out_v, g_hbm.at[b])
        out_v[...] = hvec
        pltpu.sync_copy(out_v, h_hbm.at[b])


@functools.lru_cache(maxsize=1)
def _gh_call():
    return functools.partial(
        pl.kernel,
        out_type=[
            jax.ShapeDtypeStruct((NB, 16), jnp.float32),
            jax.ShapeDtypeStruct((NB, 16), jnp.float32),
        ],
        scratch_types=[
            pltpu.VMEM((K,), jnp.int32),
            pltpu.VMEM((K,), jnp.int32),
            pltpu.VMEM((K,), jnp.int32),
            pltpu.VMEM((K,), jnp.float32),
            pltpu.VMEM((K,), jnp.float32),
            pltpu.VMEM((K,), jnp.float32),
            pltpu.VMEM((K,), jnp.float32),
            pltpu.VMEM((K,), jnp.float32),
            pltpu.VMEM((K,), jnp.float32),
            pltpu.VMEM((16,), jnp.float32),
            pltpu.SemaphoreType.DMA,
        ],
        mesh=_sc_mesh(),
        compiler_params=pltpu.CompilerParams(needs_layout_passes=False),
    )(_gh_body)


NR = N // 128


def _solve_body(g_ref, h_ref, p3_ref, bbm_ref, out_ref):
    g = g_ref[...]
    h = h_ref[...]

    def c(mat, i, j):
        k = i * 4 + j
        return mat[:, k:k + 1]

    d0 = c(g, 0, 0)
    L10 = c(g, 1, 0) / d0
    L20 = c(g, 2, 0) / d0
    L30 = c(g, 3, 0) / d0
    d1 = c(g, 1, 1) - L10 * L10 * d0
    L21 = (c(g, 2, 1) - L20 * L10 * d0) / d1
    L31 = (c(g, 3, 1) - L30 * L10 * d0) / d1
    d2_ = c(g, 2, 2) - L20 * L20 * d0 - L21 * L21 * d1
    L32 = (c(g, 3, 2) - L30 * L20 * d0 - L31 * L21 * d1) / d2_
    d3 = (c(g, 3, 3) - L30 * L30 * d0 - L31 * L31 * d1 - L32 * L32 * d2_)

    acols = []
    for j in range(4):
        h0, h1, h2, h3 = c(h, 0, j), c(h, 1, j), c(h, 2, j), c(h, 3, j)
        y0 = h0
        y1 = h1 - L10 * y0
        y2 = h2 - L20 * y0 - L21 * y1
        y3 = h3 - L30 * y0 - L31 * y1 - L32 * y2
        z0, z1, z2, z3 = y0 / d0, y1 / d1, y2 / d2_, y3 / d3
        x3 = z3
        x2 = z2 - L32 * x3
        x1 = z1 - L21 * x2 - L31 * x3
        x0 = z0 - L10 * x1 - L20 * x2 - L30 * x3
        acols.append((x0, x1, x2, x3))

    px = p3_ref[0]
    py = p3_ref[1]
    pz = p3_ref[2]
    bbm = bbm_ref[...]
    outs = []
    for j in range(3):
        acc = jnp.zeros((NR, 128), jnp.float32)
        for b in range(NB):
            a0 = acols[j][0][b, 0]
            a1 = acols[j][1][b, 0]
            a2 = acols[j][2][b, 0]
            a3 = acols[j][3][b, 0]
            val = px * a0 + py * a1 + pz * a2 + a3
            acc = jnp.where(bbm == b, val, acc)
        outs.append(acc)
    out_ref[...] = jnp.stack(outs, axis=0)


def _solve(G, H, p3, bbm):
    return pl.pallas_call(
        _solve_body,
        in_specs=[
            pl.BlockSpec((NB, 16), lambda: (0, 0)),
            pl.BlockSpec((NB, 16), lambda: (0, 0)),
            pl.BlockSpec((3, NR, 128), lambda: (0, 0, 0)),
            pl.BlockSpec((NR, 128), lambda: (0, 0)),
        ],
        out_specs=pl.BlockSpec((3, NR, 128), lambda: (0, 0, 0)),
        out_shape=jax.ShapeDtypeStruct((3, NR, 128), jnp.float32),
    )(G, H, p3, bbm)


def kernel(pos, batch, template, W1, b1, W2, b2):
    nn3, s3 = _knn(pos, template, W1, b1.reshape(1, D), W2,
                   b2.reshape(1, D))
    nn = nn3.reshape(N)
    topk = _topk(s3.reshape(1, N), batch.reshape(1, N))
    G, H = _gh_call()(topk.reshape(NB * K), pos.reshape(N * 3),
                      template.reshape(T * 3), nn)
    p3 = pos.T.reshape(3, NR, 128)
    out3 = _solve(G, H, p3, batch.reshape(NR, 128))
    return out3.reshape(3, N).T
